# baseline (speedup 1.0000x reference)
import functools

import jax
import jax.numpy as jnp
from jax import lax
from jax.experimental import pallas as pl
from jax.experimental.pallas import tpu as pltpu
from jax.experimental.pallas import tpu_sc as plsc

NC, NS, L = 2, 16, 16
NW = NC * NS
D = 64
SCALE = 8.0
INFLIGHT = 4
TC_BLK = 8192


@functools.partial(jax.jit, static_argnames=("B",))
def _sc_gather(idx_flat, table, B):
    b_per_w = B // NW
    n_grp = b_per_w // L
    mesh = plsc.VectorSubcoreMesh(core_axis_name="c", subcore_axis_name="s")

    @functools.partial(
        pl.kernel,
        out_type=jax.ShapeDtypeStruct((B, D), jnp.float32),
        mesh=mesh,
        scratch_types=[
            pltpu.VMEM((b_per_w,), jnp.int32),
            pltpu.SemaphoreType.DMA,
        ],
    )
    def k(idx_hbm, table_hbm, out_hbm, idx_v, sem_g):
        wid = lax.axis_index("s") * NC + lax.axis_index("c")
        base = wid * b_per_w
        pltpu.sync_copy(
            idx_hbm.at[pl.ds(pl.multiple_of(base, 256), b_per_w)], idx_v)

        def drain_group():
            for r in range(L):
                pltpu.make_async_copy(
                    table_hbm.at[pl.ds(0, 1)],
                    out_hbm.at[pl.ds(0, 1)],
                    sem_g,
                ).wait()

        def fire(gg, carry):
            g16 = idx_v[pl.ds(gg * L, L)]
            off = base + gg * L
            for r in range(L):
                pltpu.async_copy(
                    table_hbm.at[pl.ds(g16[r], 1)],
                    out_hbm.at[pl.ds(off + r, 1)],
                    sem_g,
                )

            @pl.when(gg >= INFLIGHT)
            def _():
                drain_group()

            return carry

        lax.fori_loop(0, n_grp, fire, 0, unroll=False)
        for _ in range(INFLIGHT):
            drain_group()

    return k(idx_flat, table)


def _tc_scale_body(idx_ref, rows_ref, o_ref):
    m = jnp.where(idx_ref[...] != 0, SCALE, 0.0).astype(jnp.float32)
    o_ref[...] = rows_ref[...] * m[:, None]


@functools.partial(jax.jit, static_argnames=("B",))
def _tc_scale(idx_flat, gathered, B):
    grid = B // TC_BLK
    return pl.pallas_call(
        _tc_scale_body,
        grid=(grid,),
        in_specs=[
            pl.BlockSpec((TC_BLK,), lambda i: (i,)),
            pl.BlockSpec((TC_BLK, D), lambda i: (i, 0)),
        ],
        out_specs=pl.BlockSpec((TC_BLK, D), lambda i: (i, 0)),
        out_shape=jax.ShapeDtypeStruct((B, D), jnp.float32),
    )(idx_flat, gathered)


def kernel(inputs, shared_weights):
    B = inputs.size
    idx_flat = inputs.reshape(B).astype(jnp.int32)
    gathered = _sc_gather(idx_flat, shared_weights, B)
    out = _tc_scale(idx_flat, gathered, B)
    return out.reshape(inputs.shape + (D,))


# speedup vs baseline: 1.0001x; 1.0001x over previous
import functools

import jax
import jax.numpy as jnp
from jax import lax
from jax.experimental import pallas as pl
from jax.experimental.pallas import tpu as pltpu
from jax.experimental.pallas import tpu_sc as plsc

NC, NS, L = 2, 16, 16
NW = NC * NS
D = 64
SCALE = 8.0
INFLIGHT = 16
TC_BLK = 8192


@functools.partial(jax.jit, static_argnames=("B",))
def _sc_gather(idx_flat, table, B):
    b_per_w = B // NW
    n_grp = b_per_w // L
    mesh = plsc.VectorSubcoreMesh(core_axis_name="c", subcore_axis_name="s")

    @functools.partial(
        pl.kernel,
        out_type=jax.ShapeDtypeStruct((B, D), jnp.float32),
        mesh=mesh,
        scratch_types=[
            pltpu.VMEM((b_per_w,), jnp.int32),
            pltpu.SemaphoreType.DMA,
        ],
    )
    def k(idx_hbm, table_hbm, out_hbm, idx_v, sem_g):
        wid = lax.axis_index("s") * NC + lax.axis_index("c")
        base = wid * b_per_w
        pltpu.sync_copy(
            idx_hbm.at[pl.ds(pl.multiple_of(base, 256), b_per_w)], idx_v)

        def drain_group():
            for r in range(L):
                pltpu.make_async_copy(
                    table_hbm.at[pl.ds(0, 1)],
                    out_hbm.at[pl.ds(0, 1)],
                    sem_g,
                ).wait()

        def fire(gg, carry):
            g16 = idx_v[pl.ds(gg * L, L)]
            off = base + gg * L
            for r in range(L):
                pltpu.async_copy(
                    table_hbm.at[pl.ds(g16[r], 1)],
                    out_hbm.at[pl.ds(off + r, 1)],
                    sem_g,
                )

            @pl.when(gg >= INFLIGHT)
            def _():
                drain_group()

            return carry

        lax.fori_loop(0, n_grp, fire, 0, unroll=False)
        for _ in range(INFLIGHT):
            drain_group()

    return k(idx_flat, table)


def _tc_scale_body(idx_ref, rows_ref, o_ref):
    m = jnp.where(idx_ref[...] != 0, SCALE, 0.0).astype(jnp.float32)
    o_ref[...] = rows_ref[...] * m[:, None]


@functools.partial(jax.jit, static_argnames=("B",))
def _tc_scale(idx_flat, gathered, B):
    grid = B // TC_BLK
    return pl.pallas_call(
        _tc_scale_body,
        grid=(grid,),
        in_specs=[
            pl.BlockSpec((TC_BLK,), lambda i: (i,)),
            pl.BlockSpec((TC_BLK, D), lambda i: (i, 0)),
        ],
        out_specs=pl.BlockSpec((TC_BLK, D), lambda i: (i, 0)),
        out_shape=jax.ShapeDtypeStruct((B, D), jnp.float32),
    )(idx_flat, gathered)


def kernel(inputs, shared_weights):
    B = inputs.size
    idx_flat = inputs.reshape(B).astype(jnp.int32)
    gathered = _sc_gather(idx_flat, shared_weights, B)
    out = _tc_scale(idx_flat, gathered, B)
    return out.reshape(inputs.shape + (D,))


# speedup vs baseline: 5.4414x; 5.4406x over previous
import functools

import jax
import jax.numpy as jnp
from jax import lax
from jax.experimental import pallas as pl
from jax.experimental.pallas import tpu as pltpu
from jax.experimental.pallas import tpu_sc as plsc

NC, NS, L = 2, 16, 16
NW = NC * NS
D = 64
SCALE = 8.0
CHUNK = 256
INFLIGHT = 4
TC_BLK = 8192


@functools.partial(jax.jit, static_argnames=("B",))
def _sc_gather(idx_flat, table, B):
    b_per_w = B // NW
    n_chunks = b_per_w // CHUNK
    mesh = plsc.VectorSubcoreMesh(core_axis_name="c", subcore_axis_name="s")

    @functools.partial(
        pl.kernel,
        out_type=jax.ShapeDtypeStruct((B, D), jnp.float32),
        mesh=mesh,
        scratch_types=[
            pltpu.VMEM((b_per_w,), jnp.int32),
            pltpu.VMEM_SHARED((NS, CHUNK, D), jnp.float32),
            pltpu.SemaphoreType.DMA,
            pltpu.SemaphoreType.DMA,
        ],
    )
    def k(idx_hbm, table_hbm, out_hbm, idx_v, shared, sem_g, sem_o):
        cid = lax.axis_index("c")
        sid = lax.axis_index("s")
        wid = sid * NC + cid
        base = wid * b_per_w
        pltpu.sync_copy(
            idx_hbm.at[pl.ds(pl.multiple_of(base, 256), b_per_w)], idx_v)

        def drain_group():
            for r in range(L):
                pltpu.make_async_copy(
                    table_hbm.at[pl.ds(0, 1)],
                    shared.at[sid, pl.ds(0, 1)],
                    sem_g,
                ).wait()

        def chunk_body(c, carry):
            def fire(gg, carry2):
                g16 = idx_v[pl.ds(c * CHUNK + gg * L, L)]
                for r in range(L):
                    pltpu.async_copy(
                        table_hbm.at[pl.ds(g16[r], 1)],
                        shared.at[sid, pl.ds(gg * L + r, 1)],
                        sem_g,
                    )

                @pl.when(gg >= INFLIGHT)
                def _():
                    drain_group()

                return carry2

            lax.fori_loop(0, CHUNK // L, fire, 0, unroll=False)
            for _ in range(INFLIGHT):
                drain_group()

            pltpu.async_copy(
                shared.at[sid],
                out_hbm.at[pl.ds(
                    pl.multiple_of(base + c * CHUNK, 256), CHUNK)],
                sem_o,
            )
            pltpu.make_async_copy(
                shared.at[sid], out_hbm.at[pl.ds(0, CHUNK)], sem_o
            ).wait()
            return carry

        lax.fori_loop(0, n_chunks, chunk_body, 0, unroll=False)

    return k(idx_flat, table)


def _tc_scale_body(idx_ref, rows_ref, o_ref):
    m = jnp.where(idx_ref[...] != 0, SCALE, 0.0).astype(jnp.float32)
    o_ref[...] = rows_ref[...] * m[:, None]


@functools.partial(jax.jit, static_argnames=("B",))
def _tc_scale(idx_flat, gathered, B):
    grid = B // TC_BLK
    return pl.pallas_call(
        _tc_scale_body,
        grid=(grid,),
        in_specs=[
            pl.BlockSpec((TC_BLK,), lambda i: (i,)),
            pl.BlockSpec((TC_BLK, D), lambda i: (i, 0)),
        ],
        out_specs=pl.BlockSpec((TC_BLK, D), lambda i: (i, 0)),
        out_shape=jax.ShapeDtypeStruct((B, D), jnp.float32),
    )(idx_flat, gathered)


def kernel(inputs, shared_weights):
    B = inputs.size
    idx_flat = inputs.reshape(B).astype(jnp.int32)
    gathered = _sc_gather(idx_flat, shared_weights, B)
    out = _tc_scale(idx_flat, gathered, B)
    return out.reshape(inputs.shape + (D,))


# speedup vs baseline: 10.1058x; 1.8572x over previous
import functools

import jax
import jax.numpy as jnp
from jax import lax
from jax.experimental import pallas as pl
from jax.experimental.pallas import tpu as pltpu
from jax.experimental.pallas import tpu_sc as plsc

NC, NS, L = 2, 16, 16
NW = NC * NS
D = 64
SCALE = 8.0
SUB = 256
CHUNK = 256
NBUF = 4


@functools.partial(jax.jit, static_argnames=("B",))
def _sc_lookup(idx_flat, table, B):
    b_per_w = B // NW
    n_chunks = b_per_w // CHUNK
    assert n_chunks % NBUF == 0 and n_chunks >= 2 * NBUF
    mesh = plsc.VectorSubcoreMesh(core_axis_name="c", subcore_axis_name="s")

    @functools.partial(
        pl.kernel,
        out_type=jax.ShapeDtypeStruct((B, D), jnp.float32),
        mesh=mesh,
        scratch_types=[
            pltpu.VMEM((b_per_w,), jnp.int32),
            pltpu.VMEM((NBUF, CHUNK, D), jnp.float32),
            pltpu.VMEM_SHARED((NS, 2, CHUNK, D), jnp.float32),
            pltpu.SemaphoreType.DMA((NBUF,)),
            pltpu.SemaphoreType.DMA((2,)),
        ],
        compiler_params=pltpu.CompilerParams(use_tc_tiling_on_sc=False),
    )
    def k(idx_hbm, table_hbm, out_hbm, idx_v, rows_v, shared, sem_g, sem_o):
        sid = lax.axis_index("s")
        wid = sid * NC + lax.axis_index("c")
        base = wid * b_per_w

        pltpu.sync_copy(idx_hbm.at[pl.ds(base, b_per_w)], idx_v)

        def stage_and_fire(c, b):
            for j in range(CHUNK // SUB):
                pltpu.async_copy(
                    table_hbm.at[idx_v.at[pl.ds(c * CHUNK + j * SUB, SUB)]],
                    rows_v.at[b, pl.ds(j * SUB, SUB)],
                    sem_g.at[b],
                )

        def drain_gathers(c, b):
            for j in range(CHUNK // SUB):
                pltpu.make_async_copy(
                    table_hbm.at[idx_v.at[pl.ds(c * CHUNK + j * SUB, SUB)]],
                    rows_v.at[b, pl.ds(j * SUB, SUB)],
                    sem_g.at[b],
                ).wait()

        def wait_outcopy(s2):
            pltpu.make_async_copy(
                shared.at[sid, s2], out_hbm.at[pl.ds(0, CHUNK)], sem_o.at[s2]
            ).wait()

        def compute(c, b):
            def grp_body(gg, carry):
                g16 = idx_v[pl.ds(c * CHUNK + gg * L, L)]
                m16 = jnp.where(g16 != 0, SCALE, 0.0).astype(jnp.float32)
                for r in range(L):
                    m = m16.at[jnp.full((L,), r, jnp.int32)].get(
                        mode="promise_in_bounds")
                    row = gg * L + r
                    for kk in range(D // L):
                        v = rows_v[b, row, pl.ds(kk * L, L)]
                        rows_v[b, row, pl.ds(kk * L, L)] = v * m
                return carry

            lax.fori_loop(0, CHUNK // L, grp_body, 0, unroll=False)

        for c in range(NBUF - 1):
            stage_and_fire(c, c)

        def outer_body(g, carry):
            for b in range(NBUF):
                c = g * NBUF + b
                drain_gathers(c, b)
                compute(c, b)
                s2 = b % 2

                @pl.when(c >= 2)
                def _():
                    wait_outcopy(s2)

                pltpu.sync_copy(rows_v.at[b], shared.at[sid, s2])
                pltpu.async_copy(
                    shared.at[sid, s2],
                    out_hbm.at[pl.ds(base + c * CHUNK, CHUNK)],
                    sem_o.at[s2],
                )
                bp = (b + NBUF - 1) % NBUF

                @pl.when(c + NBUF - 1 < n_chunks)
                def _():
                    stage_and_fire(c + NBUF - 1, bp)

            return carry

        lax.fori_loop(0, n_chunks // NBUF, outer_body, 0, unroll=False)

        for c in range(n_chunks - 2, n_chunks):
            wait_outcopy(c % 2)

    return k(idx_flat, table)


def kernel(inputs, shared_weights):
    B = inputs.size
    idx_flat = inputs.reshape(B).astype(jnp.int32)
    idx_flat = lax.optimization_barrier(idx_flat)
    out = _sc_lookup(idx_flat, shared_weights, B)
    return out.reshape(inputs.shape + (D,))
